# baseline (speedup 1.0000x reference)
import functools
import jax
import jax.numpy as jnp
from jax import lax
from jax.experimental import pallas as pl
from jax.experimental.pallas import tpu as pltpu
from jax.experimental.pallas import tpu_sc as plsc


def kernel(x_ids, table):
    bsz, seq_len = x_ids.shape
    d = table.shape[1]

    info = plsc.get_sparse_core_info()
    NC, NS = info.num_cores, info.num_subcores
    NW = NC * NS
    rows_per_w = seq_len // NW
    C = 32
    n_chunks = rows_per_w // C

    mesh = plsc.VectorSubcoreMesh(core_axis_name="c", subcore_axis_name="s")

    @functools.partial(
        pl.kernel,
        mesh=mesh,
        out_type=jax.ShapeDtypeStruct((bsz, seq_len, d), table.dtype),
        scratch_types=[pltpu.VMEM((C, d), table.dtype)],
    )
    def sc_copy(table_hbm, out_hbm, buf):
        wid = lax.axis_index("s") * NC + lax.axis_index("c")
        base = wid * rows_per_w
        for i in range(n_chunks):
            start = base + i * C
            pltpu.sync_copy(table_hbm.at[pl.ds(start, C)], buf)
            for b in range(bsz):
                pltpu.sync_copy(buf, out_hbm.at[b, pl.ds(start, C)])

    return sc_copy(table)


# speedup vs baseline: 1.0372x; 1.0372x over previous
import functools
import jax
import jax.numpy as jnp
from jax import lax
from jax.experimental import pallas as pl
from jax.experimental.pallas import tpu as pltpu
from jax.experimental.pallas import tpu_sc as plsc


def kernel(x_ids, table):
    bsz, seq_len = x_ids.shape
    d = table.shape[1]

    info = plsc.get_sparse_core_info()
    NC, NS = info.num_cores, info.num_subcores
    NW = NC * NS
    rows_per_w = seq_len // NW
    C = 32
    n_chunks = rows_per_w // C

    mesh = plsc.VectorSubcoreMesh(core_axis_name="c", subcore_axis_name="s")

    @functools.partial(
        pl.kernel,
        mesh=mesh,
        out_type=jax.ShapeDtypeStruct((bsz, seq_len, d), table.dtype),
        scratch_types=[
            pltpu.VMEM((C, d), table.dtype),
            pltpu.VMEM((C, d), table.dtype),
            pltpu.SemaphoreType.DMA,
            pltpu.SemaphoreType.DMA,
            pltpu.SemaphoreType.DMA,
        ],
    )
    def sc_copy(table_hbm, out_hbm, buf0, buf1, gsem, wsem0, wsem1):
        wid = lax.axis_index("s") * NC + lax.axis_index("c")
        base = wid * rows_per_w
        bufs = (buf0, buf1)
        wsems = (wsem0, wsem1)
        gathers = [None, None]
        scatters = [[], []]
        gathers[0] = pltpu.async_copy(table_hbm.at[pl.ds(base, C)], bufs[0], gsem)
        for i in range(n_chunks):
            k = i % 2
            gathers[k].wait()
            if i + 1 < n_chunks:
                nk = (i + 1) % 2
                for h in scatters[nk]:
                    h.wait()
                scatters[nk] = []
                gathers[nk] = pltpu.async_copy(
                    table_hbm.at[pl.ds(base + (i + 1) * C, C)], bufs[nk], gsem
                )
            start = base + i * C
            for b in range(bsz):
                scatters[k].append(
                    pltpu.async_copy(bufs[k], out_hbm.at[b, pl.ds(start, C)], wsems[k])
                )
        for k in (0, 1):
            for h in scatters[k]:
                h.wait()

    return sc_copy(table)
